# Initial kernel scaffold; baseline (speedup 1.0000x reference)
import jax
import jax.numpy as jnp
from jax.experimental import pallas as pl


def kernel(x, edge_index, edge_weight, W1, b1, W2, b2):
    raise NotImplementedError("write your pallas kernel here")


# speedup vs baseline: 5.5134x; 5.5134x over previous
import functools

import jax
import jax.numpy as jnp
from jax import lax
from jax.experimental import pallas as pl
from jax.experimental.pallas import tpu as pltpu
from jax.experimental.pallas import tpu_sc as plsc

N = 10000
NP = 10240
E = 160000
EP = 163840
IN_CH = 128
HID = 64
OUT = 128
T = 4
NC = 2
NS = 16
TILE_N = NP // NS
EB = 128
EIT = EP // NS // EB
EITD = EP // (NC * NS) // EB
NBT = 1280
NBLK = NP // NBT

_mesh = plsc.VectorSubcoreMesh(core_axis_name="c", subcore_axis_name="s")


def _deg_body(dst_hbm, w_hbm, out_hbm, dst_v, wsp_v, rows_v, deg_sh):
    cid = lax.axis_index("c")
    sid = lax.axis_index("s")
    wid = sid * NC + cid
    pltpu.sync_copy(dst_hbm.at[wid], dst_v)

    def zrow(r, c):
        for g in range(8):
            rows_v[r, pl.ds(g * 16, 16)] = jnp.zeros((16,), jnp.float32)
        return c
    lax.fori_loop(0, EB, zrow, 0)
    for k in range(TILE_N // EB):
        pltpu.sync_copy(rows_v, deg_sh.at[pl.ds(sid * TILE_N + k * EB, EB)])
    plsc.subcore_barrier()

    lane = lax.iota(jnp.int32, 16)

    def eit(i, c):
        pltpu.sync_copy(w_hbm.at[wid, i], wsp_v)

        def row(r, c2):
            rows_v[r, pl.ds(0, 16)] = jnp.where(lane == 0, wsp_v[r, :], 0.0)
            return c2
        lax.fori_loop(0, EB, row, 0)
        pltpu.sync_copy(rows_v, deg_sh.at[dst_v.at[i]], add=True)
        return c
    lax.fori_loop(0, EITD, eit, 0)
    plsc.subcore_barrier()

    for cc in range(NC):
        @pl.when(cid == cc)
        def _():
            pltpu.sync_copy(deg_sh.at[pl.ds(sid * TILE_N, TILE_N)],
                            out_hbm.at[cc, pl.ds(sid * TILE_N, TILE_N)])


_deg_kernel = functools.partial(
    pl.kernel, _deg_body,
    out_type=jax.ShapeDtypeStruct((NC, NP, 128), jnp.float32),
    scratch_types=[
        pltpu.VMEM((EITD, EB), jnp.int32),
        pltpu.VMEM((EB, 16), jnp.float32),
        pltpu.VMEM((EB, 128), jnp.float32),
        pltpu.VMEM_SHARED((NP, 128), jnp.float32),
    ],
    mesh=_mesh,
)()


def _chunk_pass(width, table_hbm, src_hbm, dst_hbm, w_hbm, out_slot,
                src_v, dst_v, wsp_v, rows_v, agg_sh, sem, sid):
    ng = width // 16

    def zrow(r, c):
        for g in range(ng):
            rows_v[r, pl.ds(g * 16, 16)] = jnp.zeros((16,), jnp.float32)
        return c
    lax.fori_loop(0, EB, zrow, 0)
    for k in range(TILE_N // EB):
        pltpu.sync_copy(rows_v, agg_sh.at[pl.ds(sid * TILE_N + k * EB, EB)])
    plsc.subcore_barrier()

    def eit(i, c):
        pltpu.sync_copy(src_hbm.at[sid, i], src_v)
        pltpu.sync_copy(w_hbm.at[sid, i], wsp_v)
        pltpu.async_copy(table_hbm.at[src_v.at[0]], rows_v, sem).wait()

        def scale(r, c2):
            wsrow = wsp_v[r, :]
            for g in range(ng):
                sl = pl.ds(g * 16, 16)
                rows_v[r, sl] = rows_v[r, sl] * wsrow
            return c2
        lax.fori_loop(0, EB, scale, 0)
        pltpu.sync_copy(rows_v, agg_sh.at[dst_v.at[i]], add=True)
        return c
    lax.fori_loop(0, EIT, eit, 0)
    plsc.subcore_barrier()
    pltpu.sync_copy(agg_sh.at[pl.ds(sid * TILE_N, TILE_N)],
                    out_slot.at[pl.ds(sid * TILE_N, TILE_N)])
    plsc.subcore_barrier()


def _mk_edge_kernel(ntab):
    width = 128
    npass = ntab // NC

    def body(*refs):
        tables = refs[:ntab]
        src_hbm, dst_hbm, w_hbm, out_hbm = refs[ntab:ntab + 4]
        src_v, dst_v, w_v, rows_v, agg_sh, sem = refs[ntab + 4:]
        cid = lax.axis_index("c")
        sid = lax.axis_index("s")
        pltpu.sync_copy(dst_hbm.at[sid], dst_v)
        for cc in range(NC):
            @pl.when(cid == cc)
            def _():
                for j in range(npass):
                    ch = cc * npass + j
                    _chunk_pass(width, tables[ch], src_hbm, dst_hbm, w_hbm,
                                out_hbm.at[ch], src_v, dst_v, w_v, rows_v,
                                agg_sh, sem, sid)

    return functools.partial(
        pl.kernel, body,
        out_type=jax.ShapeDtypeStruct((ntab, NP, width), jnp.float32),
        scratch_types=[
            pltpu.VMEM((1, EB), jnp.int32),
            pltpu.VMEM((EIT, EB), jnp.int32),
            pltpu.VMEM((EB, 16), jnp.float32),
            pltpu.VMEM((EB, width), jnp.float32),
            pltpu.VMEM_SHARED((NP, width), jnp.float32),
            pltpu.SemaphoreType.DMA,
        ],
        mesh=_mesh,
    )()


_l1_kernel = _mk_edge_kernel(2)
_l2_kernel = _mk_edge_kernel(4)


def _dis_of(d_ref):
    deg = d_ref[0, :, 0:1] + d_ref[1, :, 0:1] + 1.0
    return jnp.where(deg > 0, lax.rsqrt(jnp.maximum(deg, 1e-12)), 0.0)


def _c1_body(x_ref, w_ref, d_ref, o_ref):
    dis = _dis_of(d_ref)
    for t2 in range(2):
        xw = lax.dot_general(x_ref[t2], w_ref[...], (((0,), (0,)), ((), ())),
                             preferred_element_type=jnp.float32)
        o_ref[0, :, t2 * HID:(t2 + 1) * HID] = xw * dis


def _c1(xpt, w1, degp):
    return pl.pallas_call(
        _c1_body,
        grid=(T // 2, NBLK),
        in_specs=[
            pl.BlockSpec((2, IN_CH, NBT), lambda p, n: (p, 0, n)),
            pl.BlockSpec((IN_CH, HID), lambda p, n: (0, 0)),
            pl.BlockSpec((NC, NBT, 128), lambda p, n: (0, n, 0)),
        ],
        out_specs=pl.BlockSpec((1, NBT, 128), lambda p, n: (p, n, 0)),
        out_shape=jax.ShapeDtypeStruct((2, NP, 128), jnp.float32),
    )(xpt, w1, degp)


def _c2_body(a_ref, y_ref, d_ref, w_ref, b_ref, o_ref):
    dis = _dis_of(d_ref)
    for t2 in range(2):
        sl = slice(t2 * HID, (t2 + 1) * HID)
        h = dis * (a_ref[0, :, sl] + y_ref[0, :, sl]) + b_ref[...]
        xw = lax.dot_general(h, w_ref[...], (((1,), (0,)), ((), ())),
                             preferred_element_type=jnp.float32)
        o_ref[t2] = xw * dis


def _c2(agg1, y1, degp, w2, b1):
    return pl.pallas_call(
        _c2_body,
        grid=(T // 2, NBLK),
        in_specs=[
            pl.BlockSpec((1, NBT, 128), lambda p, n: (p, n, 0)),
            pl.BlockSpec((1, NBT, 128), lambda p, n: (p, n, 0)),
            pl.BlockSpec((NC, NBT, 128), lambda p, n: (0, n, 0)),
            pl.BlockSpec((HID, OUT), lambda p, n: (0, 0)),
            pl.BlockSpec((1, HID), lambda p, n: (0, 0)),
        ],
        out_specs=pl.BlockSpec((2, NBT, OUT), lambda p, n: (p, n, 0)),
        out_shape=jax.ShapeDtypeStruct((T, NP, OUT), jnp.float32),
    )(agg1, y1, degp, w2, b1)


def _c3_body(a_ref, y_ref, d_ref, b_ref, o_ref):
    dis = _dis_of(d_ref)
    o = jnp.tanh(dis * (a_ref[0] + y_ref[0]) + b_ref[...])
    o_ref[...] = o.T[None]


def _c3(agg2, y2, degp, b2):
    return pl.pallas_call(
        _c3_body,
        grid=(T, NBLK),
        in_specs=[
            pl.BlockSpec((1, NBT, OUT), lambda t, n: (t, n, 0)),
            pl.BlockSpec((1, NBT, OUT), lambda t, n: (t, n, 0)),
            pl.BlockSpec((NC, NBT, 128), lambda t, n: (0, n, 0)),
            pl.BlockSpec((1, OUT), lambda t, n: (0, 0)),
        ],
        out_specs=pl.BlockSpec((1, OUT, NBT), lambda t, n: (t, 0, n)),
        out_shape=jax.ShapeDtypeStruct((T, OUT, NP), jnp.float32),
    )(agg2, y2, degp, b2)


def kernel(x, edge_index, edge_weight, W1, b1, W2, b2):
    src, dst = edge_index[0], edge_index[1]
    pad = EP - E
    srcp = jnp.pad(src, (0, pad))
    dstp = jnp.pad(dst, (0, pad))
    wp = jnp.pad(edge_weight, (0, pad))
    src3 = srcp.reshape(NS, EIT, 1, EB)
    dst3 = dstp.reshape(NS, EIT, EB)
    wsp = jnp.broadcast_to(wp[:, None], (EP, 16))
    w3 = wsp.reshape(NS, EIT, EB, 16)
    dst3d = dstp.reshape(NC * NS, EITD, EB)
    w3d = wsp.reshape(NC * NS, EITD, EB, 16)
    xpt = jnp.pad(x[0].transpose(1, 0, 2), ((0, 0), (0, 0), (0, NP - N)))

    degp = _deg_kernel(dst3d, w3d)
    y1 = _c1(xpt, W1, degp)
    agg1 = _l1_kernel(y1[0], y1[1], src3, dst3, w3)
    y2 = _c2(agg1, y1, degp, W2, b1.reshape(1, HID))
    agg2 = _l2_kernel(y2[0], y2[1], y2[2], y2[3], src3, dst3, w3)
    outp = _c3(agg2, y2, degp, b2.reshape(1, OUT))
    return outp[:, :, :N].transpose(1, 0, 2)[None]


# speedup vs baseline: 6.2892x; 1.1407x over previous
import functools

import jax
import jax.numpy as jnp
from jax import lax
from jax.experimental import pallas as pl
from jax.experimental.pallas import tpu as pltpu
from jax.experimental.pallas import tpu_sc as plsc

N = 10000
NP = 10240
E = 160000
EP = 163840
IN_CH = 128
HID = 64
OUT = 128
T = 4
NC = 2
NS = 16
TILE_N = NP // NS
EB = 128
EIT = EP // NS // EB
EBE = 64
EITE = EP // NS // EBE
EITD = EP // (NC * NS) // EB
NBT = 1280
NBLK = NP // NBT

_mesh = plsc.VectorSubcoreMesh(core_axis_name="c", subcore_axis_name="s")


def _deg_body(dst_hbm, w_hbm, out_hbm, dst_v, wsp_v, rows_v, deg_sh):
    cid = lax.axis_index("c")
    sid = lax.axis_index("s")
    wid = sid * NC + cid
    pltpu.sync_copy(dst_hbm.at[wid], dst_v)

    def zrow(r, c):
        for g in range(8):
            rows_v[r, pl.ds(g * 16, 16)] = jnp.zeros((16,), jnp.float32)
        return c
    lax.fori_loop(0, EB, zrow, 0)
    for k in range(TILE_N // EB):
        pltpu.sync_copy(rows_v, deg_sh.at[pl.ds(sid * TILE_N + k * EB, EB)])
    plsc.subcore_barrier()

    lane = lax.iota(jnp.int32, 16)

    def eit(i, c):
        pltpu.sync_copy(w_hbm.at[wid, i], wsp_v)

        def row(r, c2):
            rows_v[r, pl.ds(0, 16)] = jnp.where(lane == 0, wsp_v[r, :], 0.0)
            return c2
        lax.fori_loop(0, EB, row, 0)
        pltpu.sync_copy(rows_v, deg_sh.at[dst_v.at[i]], add=True)
        return c
    lax.fori_loop(0, EITD, eit, 0)
    plsc.subcore_barrier()

    for cc in range(NC):
        @pl.when(cid == cc)
        def _():
            pltpu.sync_copy(deg_sh.at[pl.ds(sid * TILE_N, TILE_N)],
                            out_hbm.at[cc, pl.ds(sid * TILE_N, TILE_N)])


_deg_kernel = functools.partial(
    pl.kernel, _deg_body,
    out_type=jax.ShapeDtypeStruct((NC, NP, 128), jnp.float32),
    scratch_types=[
        pltpu.VMEM((EITD, EB), jnp.int32),
        pltpu.VMEM((EB, 16), jnp.float32),
        pltpu.VMEM((EB, 128), jnp.float32),
        pltpu.VMEM_SHARED((NP, 128), jnp.float32),
    ],
    mesh=_mesh,
)()


def _chunk_pass(width, table_hbm, src_hbm, dst_hbm, w_hbm, out_slot,
                srcs, dst_v, wsps, rowss, agg_sh, sems, sid):
    ng = width // 16
    rows_a, rows_b = rowss

    def zrow(r, c):
        for g in range(ng):
            rows_a[r, pl.ds(g * 16, 16)] = jnp.zeros((16,), jnp.float32)
        return c
    lax.fori_loop(0, EBE, zrow, 0)
    for k in range(TILE_N // EBE):
        pltpu.sync_copy(rows_a,
                        agg_sh.at[pl.ds(sid * TILE_N + k * EBE, EBE)])
    plsc.subcore_barrier()

    def scale_scatter(rows_v, wsp_v, i):
        def scale(r, c2):
            wsrow = wsp_v[r, :]
            for g in range(ng):
                sl = pl.ds(g * 16, 16)
                rows_v[r, sl] = rows_v[r, sl] * wsrow
            return c2
        lax.fori_loop(0, EBE, scale, 0)
        half = EITE // 2
        loc = jnp.where(i < half, i, i - half)
        pltpu.sync_copy(rows_v, agg_sh.at[dst_v.at[loc]], add=True)

    pltpu.sync_copy(dst_hbm.at[sid, pl.ds(0, EITE // 2)], dst_v)
    pltpu.sync_copy(src_hbm.at[sid, 0], srcs[0])
    pltpu.async_copy(table_hbm.at[srcs[0].at[0]], rows_a, sems[0])

    def body(k, c):
        i0 = 2 * k
        i1 = 2 * k + 1

        @pl.when(i0 == EITE // 2)
        def _():
            pltpu.sync_copy(dst_hbm.at[sid, pl.ds(EITE // 2, EITE // 2)],
                            dst_v)
        pltpu.sync_copy(src_hbm.at[sid, i1], srcs[1])
        pltpu.sync_copy(w_hbm.at[sid, i0], wsps[0])
        pltpu.make_async_copy(table_hbm.at[srcs[0].at[0]], rows_a,
                              sems[0]).wait()
        pltpu.async_copy(table_hbm.at[srcs[1].at[0]], rows_b, sems[1])
        scale_scatter(rows_a, wsps[0], i0)
        @pl.when(i1 + 1 < EITE)
        def _():
            pltpu.sync_copy(src_hbm.at[sid, i1 + 1], srcs[0])
        pltpu.sync_copy(w_hbm.at[sid, i1], wsps[1])
        pltpu.make_async_copy(table_hbm.at[srcs[1].at[0]], rows_b,
                              sems[1]).wait()

        @pl.when(i1 + 1 < EITE)
        def _():
            pltpu.async_copy(table_hbm.at[srcs[0].at[0]], rows_a, sems[0])
        scale_scatter(rows_b, wsps[1], i1)
        return c
    lax.fori_loop(0, EITE // 2, body, 0)
    plsc.subcore_barrier()
    pltpu.sync_copy(agg_sh.at[pl.ds(sid * TILE_N, TILE_N)],
                    out_slot.at[pl.ds(sid * TILE_N, TILE_N)])
    plsc.subcore_barrier()


def _mk_edge_kernel(ntab):
    width = 128
    npass = ntab // NC

    def body(*refs):
        tables = refs[:ntab]
        src_hbm, dst_hbm, w_hbm, out_hbm = refs[ntab:ntab + 4]
        (src_a, src_b, dst_v, wsp_a, wsp_b, rows_a, rows_b, agg_sh,
         sem_a, sem_b) = refs[ntab + 4:]
        cid = lax.axis_index("c")
        sid = lax.axis_index("s")
        for cc in range(NC):
            @pl.when(cid == cc)
            def _():
                for j in range(npass):
                    ch = cc * npass + j
                    _chunk_pass(width, tables[ch], src_hbm, dst_hbm, w_hbm,
                                out_hbm.at[ch], (src_a, src_b), dst_v,
                                (wsp_a, wsp_b), (rows_a, rows_b), agg_sh,
                                (sem_a, sem_b), sid)

    return functools.partial(
        pl.kernel, body,
        out_type=jax.ShapeDtypeStruct((ntab, NP, width), jnp.float32),
        scratch_types=[
            pltpu.VMEM((1, EBE), jnp.int32),
            pltpu.VMEM((1, EBE), jnp.int32),
            pltpu.VMEM((EITE // 2, EBE), jnp.int32),
            pltpu.VMEM((EBE, 16), jnp.float32),
            pltpu.VMEM((EBE, 16), jnp.float32),
            pltpu.VMEM((EBE, width), jnp.float32),
            pltpu.VMEM((EBE, width), jnp.float32),
            pltpu.VMEM_SHARED((NP, width), jnp.float32),
            pltpu.SemaphoreType.DMA,
            pltpu.SemaphoreType.DMA,
        ],
        mesh=_mesh,
    )()


_l1_kernel = _mk_edge_kernel(2)
_l2_kernel = _mk_edge_kernel(4)


def _dis_of(d_ref):
    deg = d_ref[0, :, 0:1] + d_ref[1, :, 0:1] + 1.0
    return jnp.where(deg > 0, lax.rsqrt(jnp.maximum(deg, 1e-12)), 0.0)


def _c1_body(x_ref, w_ref, d_ref, o_ref):
    dis = _dis_of(d_ref)
    for t2 in range(2):
        xw = lax.dot_general(x_ref[t2], w_ref[...], (((0,), (0,)), ((), ())),
                             preferred_element_type=jnp.float32)
        o_ref[0, :, t2 * HID:(t2 + 1) * HID] = xw * dis


def _c1(xpt, w1, degp):
    return pl.pallas_call(
        _c1_body,
        grid=(T // 2, NBLK),
        in_specs=[
            pl.BlockSpec((2, IN_CH, NBT), lambda p, n: (p, 0, n)),
            pl.BlockSpec((IN_CH, HID), lambda p, n: (0, 0)),
            pl.BlockSpec((NC, NBT, 128), lambda p, n: (0, n, 0)),
        ],
        out_specs=pl.BlockSpec((1, NBT, 128), lambda p, n: (p, n, 0)),
        out_shape=jax.ShapeDtypeStruct((2, NP, 128), jnp.float32),
    )(xpt, w1, degp)


def _c2_body(a_ref, y_ref, d_ref, w_ref, b_ref, o_ref):
    dis = _dis_of(d_ref)
    for t2 in range(2):
        sl = slice(t2 * HID, (t2 + 1) * HID)
        h = dis * (a_ref[0, :, sl] + y_ref[0, :, sl]) + b_ref[...]
        xw = lax.dot_general(h, w_ref[...], (((1,), (0,)), ((), ())),
                             preferred_element_type=jnp.float32)
        o_ref[t2] = xw * dis


def _c2(agg1, y1, degp, w2, b1):
    return pl.pallas_call(
        _c2_body,
        grid=(T // 2, NBLK),
        in_specs=[
            pl.BlockSpec((1, NBT, 128), lambda p, n: (p, n, 0)),
            pl.BlockSpec((1, NBT, 128), lambda p, n: (p, n, 0)),
            pl.BlockSpec((NC, NBT, 128), lambda p, n: (0, n, 0)),
            pl.BlockSpec((HID, OUT), lambda p, n: (0, 0)),
            pl.BlockSpec((1, HID), lambda p, n: (0, 0)),
        ],
        out_specs=pl.BlockSpec((2, NBT, OUT), lambda p, n: (p, n, 0)),
        out_shape=jax.ShapeDtypeStruct((T, NP, OUT), jnp.float32),
    )(agg1, y1, degp, w2, b1)


def _c3_body(a_ref, y_ref, d_ref, b_ref, o_ref):
    dis = _dis_of(d_ref)
    o = jnp.tanh(dis * (a_ref[0] + y_ref[0]) + b_ref[...])
    o_ref[...] = o.T[None]


def _c3(agg2, y2, degp, b2):
    return pl.pallas_call(
        _c3_body,
        grid=(T, NBLK),
        in_specs=[
            pl.BlockSpec((1, NBT, OUT), lambda t, n: (t, n, 0)),
            pl.BlockSpec((1, NBT, OUT), lambda t, n: (t, n, 0)),
            pl.BlockSpec((NC, NBT, 128), lambda t, n: (0, n, 0)),
            pl.BlockSpec((1, OUT), lambda t, n: (0, 0)),
        ],
        out_specs=pl.BlockSpec((1, OUT, NBT), lambda t, n: (t, 0, n)),
        out_shape=jax.ShapeDtypeStruct((T, OUT, NP), jnp.float32),
    )(agg2, y2, degp, b2)


def kernel(x, edge_index, edge_weight, W1, b1, W2, b2):
    src, dst = edge_index[0], edge_index[1]
    pad = EP - E
    srcp = jnp.pad(src, (0, pad))
    dstp = jnp.pad(dst, (0, pad))
    wp = jnp.pad(edge_weight, (0, pad))
    src3 = srcp.reshape(NS, EITE, 1, EBE)
    dst3 = dstp.reshape(NS, EITE, EBE)
    wsp = jnp.broadcast_to(wp[:, None], (EP, 16))
    w3 = wsp.reshape(NS, EITE, EBE, 16)
    dst3d = dstp.reshape(NC * NS, EITD, EB)
    w3d = wsp.reshape(NC * NS, EITD, EB, 16)
    xpt = jnp.pad(x[0].transpose(1, 0, 2), ((0, 0), (0, 0), (0, NP - N)))

    degp = _deg_kernel(dst3d, w3d)
    y1 = _c1(xpt, W1, degp)
    agg1 = _l1_kernel(y1[0], y1[1], src3, dst3, w3)
    y2 = _c2(agg1, y1, degp, W2, b1.reshape(1, HID))
    agg2 = _l2_kernel(y2[0], y2[1], y2[2], y2[3], src3, dst3, w3)
    outp = _c3(agg2, y2, degp, b2.reshape(1, OUT))
    return outp[:, :, :N].transpose(1, 0, 2)[None]


# speedup vs baseline: 6.6529x; 1.0578x over previous
import functools

import jax
import jax.numpy as jnp
from jax import lax
from jax.experimental import pallas as pl
from jax.experimental.pallas import tpu as pltpu
from jax.experimental.pallas import tpu_sc as plsc

N = 10000
NP = 10240
E = 160000
EP = 163840
IN_CH = 128
HID = 64
OUT = 128
T = 4
NC = 2
NS = 16
TILE_N = NP // NS
EB = 128
EIT = EP // NS // EB
EBE = 64
EITE = EP // NS // EBE
EITD = EP // (NC * NS) // EB
NBT = 1280
NBLK = NP // NBT

_mesh = plsc.VectorSubcoreMesh(core_axis_name="c", subcore_axis_name="s")


def _deg_body(dst_hbm, w_hbm, out_hbm, dst_v, wsp_v, rows_v, deg_sh):
    cid = lax.axis_index("c")
    sid = lax.axis_index("s")
    wid = sid * NC + cid
    pltpu.sync_copy(dst_hbm.at[wid], dst_v)

    def zrow(r, c):
        for g in range(8):
            rows_v[r, pl.ds(g * 16, 16)] = jnp.zeros((16,), jnp.float32)
        return c
    lax.fori_loop(0, EB, zrow, 0)
    for k in range(TILE_N // EB):
        pltpu.sync_copy(rows_v, deg_sh.at[pl.ds(sid * TILE_N + k * EB, EB)])
    plsc.subcore_barrier()

    lane = lax.iota(jnp.int32, 16)

    def eit(i, c):
        pltpu.sync_copy(w_hbm.at[wid, i], wsp_v)

        def row(r, c2):
            rows_v[r, pl.ds(0, 16)] = jnp.where(lane == 0, wsp_v[r, :], 0.0)
            return c2
        lax.fori_loop(0, EB, row, 0)
        pltpu.sync_copy(rows_v, deg_sh.at[dst_v.at[i]], add=True)
        return c
    lax.fori_loop(0, EITD, eit, 0)
    plsc.subcore_barrier()

    for cc in range(NC):
        @pl.when(cid == cc)
        def _():
            pltpu.sync_copy(deg_sh.at[pl.ds(sid * TILE_N, TILE_N)],
                            out_hbm.at[cc, pl.ds(sid * TILE_N, TILE_N)])


_deg_kernel = functools.partial(
    pl.kernel, _deg_body,
    out_type=jax.ShapeDtypeStruct((NC, NP, 128), jnp.float32),
    scratch_types=[
        pltpu.VMEM((EITD, EB), jnp.int32),
        pltpu.VMEM((EB, 16), jnp.float32),
        pltpu.VMEM((EB, 128), jnp.float32),
        pltpu.VMEM_SHARED((NP, 128), jnp.float32),
    ],
    mesh=_mesh,
)()


def _chunk_pass(width, table_hbm, src_hbm, dst_hbm, w_hbm, out_slot,
                srcs, dst_v, wsps, rowss, agg_sh, sems, sc_sems, sid):
    ng = width // 16
    rows_a, rows_b = rowss

    def zrow(r, c):
        for g in range(ng):
            rows_a[r, pl.ds(g * 16, 16)] = jnp.zeros((16,), jnp.float32)
        return c
    lax.fori_loop(0, EBE, zrow, 0)
    for k in range(TILE_N // EBE):
        pltpu.sync_copy(rows_a,
                        agg_sh.at[pl.ds(sid * TILE_N + k * EBE, EBE)])
    plsc.subcore_barrier()

    def drain_scatter(rows_v, j):
        pltpu.make_async_copy(rows_v, agg_sh.at[dst_v.at[0]],
                              sc_sems[j]).wait()

    def scale_scatter(rows_v, wsp_v, i, j):
        def scale(r, c2):
            wsrow = wsp_v[r, :]
            for g in range(ng):
                sl = pl.ds(g * 16, 16)
                rows_v[r, sl] = rows_v[r, sl] * wsrow
            return c2
        lax.fori_loop(0, EBE, scale, 0)
        half = EITE // 2
        loc = jnp.where(i < half, i, i - half)
        pltpu.async_copy(rows_v, agg_sh.at[dst_v.at[loc]], sc_sems[j],
                         add=True)

    pltpu.sync_copy(dst_hbm.at[sid, pl.ds(0, EITE // 2)], dst_v)
    pltpu.sync_copy(src_hbm.at[sid, 0], srcs[0])
    pltpu.async_copy(table_hbm.at[srcs[0].at[0]], rows_a, sems[0])

    def body(k, c):
        i0 = 2 * k
        i1 = 2 * k + 1

        @pl.when(i0 == EITE // 2)
        def _():
            pltpu.sync_copy(dst_hbm.at[sid, pl.ds(EITE // 2, EITE // 2)],
                            dst_v)
        pltpu.sync_copy(src_hbm.at[sid, i1], srcs[1])
        pltpu.sync_copy(w_hbm.at[sid, i0], wsps[0])
        pltpu.make_async_copy(table_hbm.at[srcs[0].at[0]], rows_a,
                              sems[0]).wait()
        @pl.when(k > 0)
        def _():
            drain_scatter(rows_b, 1)
        pltpu.async_copy(table_hbm.at[srcs[1].at[0]], rows_b, sems[1])
        scale_scatter(rows_a, wsps[0], i0, 0)
        @pl.when(i1 + 1 < EITE)
        def _():
            pltpu.sync_copy(src_hbm.at[sid, i1 + 1], srcs[0])
        pltpu.sync_copy(w_hbm.at[sid, i1], wsps[1])
        pltpu.make_async_copy(table_hbm.at[srcs[1].at[0]], rows_b,
                              sems[1]).wait()
        drain_scatter(rows_a, 0)

        @pl.when(i1 + 1 < EITE)
        def _():
            pltpu.async_copy(table_hbm.at[srcs[0].at[0]], rows_a, sems[0])
        scale_scatter(rows_b, wsps[1], i1, 1)
        return c
    lax.fori_loop(0, EITE // 2, body, 0)
    drain_scatter(rows_b, 1)
    plsc.subcore_barrier()
    pltpu.sync_copy(agg_sh.at[pl.ds(sid * TILE_N, TILE_N)],
                    out_slot.at[pl.ds(sid * TILE_N, TILE_N)])
    plsc.subcore_barrier()


def _mk_edge_kernel(ntab):
    width = 128
    npass = ntab // NC

    def body(*refs):
        tables = refs[:ntab]
        src_hbm, dst_hbm, w_hbm, out_hbm = refs[ntab:ntab + 4]
        (src_a, src_b, dst_v, wsp_a, wsp_b, rows_a, rows_b, agg_sh,
         sem_a, sem_b, sem_sa, sem_sb) = refs[ntab + 4:]
        cid = lax.axis_index("c")
        sid = lax.axis_index("s")
        for cc in range(NC):
            @pl.when(cid == cc)
            def _():
                for j in range(npass):
                    ch = cc * npass + j
                    _chunk_pass(width, tables[ch], src_hbm, dst_hbm, w_hbm,
                                out_hbm.at[ch], (src_a, src_b), dst_v,
                                (wsp_a, wsp_b), (rows_a, rows_b), agg_sh,
                                (sem_a, sem_b), (sem_sa, sem_sb), sid)

    return functools.partial(
        pl.kernel, body,
        out_type=jax.ShapeDtypeStruct((ntab, NP, width), jnp.float32),
        scratch_types=[
            pltpu.VMEM((1, EBE), jnp.int32),
            pltpu.VMEM((1, EBE), jnp.int32),
            pltpu.VMEM((EITE // 2, EBE), jnp.int32),
            pltpu.VMEM((EBE, 16), jnp.float32),
            pltpu.VMEM((EBE, 16), jnp.float32),
            pltpu.VMEM((EBE, width), jnp.float32),
            pltpu.VMEM((EBE, width), jnp.float32),
            pltpu.VMEM_SHARED((NP, width), jnp.float32),
            pltpu.SemaphoreType.DMA,
            pltpu.SemaphoreType.DMA,
            pltpu.SemaphoreType.DMA,
            pltpu.SemaphoreType.DMA,
        ],
        mesh=_mesh,
    )()


_l1_kernel = _mk_edge_kernel(2)
_l2_kernel = _mk_edge_kernel(4)


def _dis_of(d_ref):
    deg = d_ref[0, :, 0:1] + d_ref[1, :, 0:1] + 1.0
    return jnp.where(deg > 0, lax.rsqrt(jnp.maximum(deg, 1e-12)), 0.0)


def _c1_body(x_ref, w_ref, d_ref, o_ref):
    dis = _dis_of(d_ref)
    for t2 in range(2):
        xw = lax.dot_general(x_ref[t2], w_ref[...], (((0,), (0,)), ((), ())),
                             preferred_element_type=jnp.float32)
        o_ref[0, :, t2 * HID:(t2 + 1) * HID] = xw * dis


def _c1(xpt, w1, degp):
    return pl.pallas_call(
        _c1_body,
        grid=(T // 2, NBLK),
        in_specs=[
            pl.BlockSpec((2, IN_CH, NBT), lambda p, n: (p, 0, n)),
            pl.BlockSpec((IN_CH, HID), lambda p, n: (0, 0)),
            pl.BlockSpec((NC, NBT, 128), lambda p, n: (0, n, 0)),
        ],
        out_specs=pl.BlockSpec((1, NBT, 128), lambda p, n: (p, n, 0)),
        out_shape=jax.ShapeDtypeStruct((2, NP, 128), jnp.float32),
    )(xpt, w1, degp)


def _c2_body(a_ref, y_ref, d_ref, w_ref, b_ref, o_ref):
    dis = _dis_of(d_ref)
    for t2 in range(2):
        sl = slice(t2 * HID, (t2 + 1) * HID)
        h = dis * (a_ref[0, :, sl] + y_ref[0, :, sl]) + b_ref[...]
        xw = lax.dot_general(h, w_ref[...], (((1,), (0,)), ((), ())),
                             preferred_element_type=jnp.float32)
        o_ref[t2] = xw * dis


def _c2(agg1, y1, degp, w2, b1):
    return pl.pallas_call(
        _c2_body,
        grid=(T // 2, NBLK),
        in_specs=[
            pl.BlockSpec((1, NBT, 128), lambda p, n: (p, n, 0)),
            pl.BlockSpec((1, NBT, 128), lambda p, n: (p, n, 0)),
            pl.BlockSpec((NC, NBT, 128), lambda p, n: (0, n, 0)),
            pl.BlockSpec((HID, OUT), lambda p, n: (0, 0)),
            pl.BlockSpec((1, HID), lambda p, n: (0, 0)),
        ],
        out_specs=pl.BlockSpec((2, NBT, OUT), lambda p, n: (p, n, 0)),
        out_shape=jax.ShapeDtypeStruct((T, NP, OUT), jnp.float32),
    )(agg1, y1, degp, w2, b1)


def _c3_body(a_ref, y_ref, d_ref, b_ref, o_ref):
    dis = _dis_of(d_ref)
    o = jnp.tanh(dis * (a_ref[0] + y_ref[0]) + b_ref[...])
    o_ref[...] = o.T[None]


def _c3(agg2, y2, degp, b2):
    return pl.pallas_call(
        _c3_body,
        grid=(T, NBLK),
        in_specs=[
            pl.BlockSpec((1, NBT, OUT), lambda t, n: (t, n, 0)),
            pl.BlockSpec((1, NBT, OUT), lambda t, n: (t, n, 0)),
            pl.BlockSpec((NC, NBT, 128), lambda t, n: (0, n, 0)),
            pl.BlockSpec((1, OUT), lambda t, n: (0, 0)),
        ],
        out_specs=pl.BlockSpec((1, OUT, NBT), lambda t, n: (t, 0, n)),
        out_shape=jax.ShapeDtypeStruct((T, OUT, NP), jnp.float32),
    )(agg2, y2, degp, b2)


def kernel(x, edge_index, edge_weight, W1, b1, W2, b2):
    src, dst = edge_index[0], edge_index[1]
    pad = EP - E
    srcp = jnp.pad(src, (0, pad))
    dstp = jnp.pad(dst, (0, pad))
    wp = jnp.pad(edge_weight, (0, pad))
    src3 = srcp.reshape(NS, EITE, 1, EBE)
    dst3 = dstp.reshape(NS, EITE, EBE)
    wsp = jnp.broadcast_to(wp[:, None], (EP, 16))
    w3 = wsp.reshape(NS, EITE, EBE, 16)
    dst3d = dstp.reshape(NC * NS, EITD, EB)
    w3d = wsp.reshape(NC * NS, EITD, EB, 16)
    xpt = jnp.pad(x[0].transpose(1, 0, 2), ((0, 0), (0, 0), (0, NP - N)))

    degp = _deg_kernel(dst3d, w3d)
    y1 = _c1(xpt, W1, degp)
    agg1 = _l1_kernel(y1[0], y1[1], src3, dst3, w3)
    y2 = _c2(agg1, y1, degp, W2, b1.reshape(1, HID))
    agg2 = _l2_kernel(y2[0], y2[1], y2[2], y2[3], src3, dst3, w3)
    outp = _c3(agg2, y2, degp, b2.reshape(1, OUT))
    return outp[:, :, :N].transpose(1, 0, 2)[None]
